# baseline (speedup 1.0000x reference)
import functools

import jax
import jax.numpy as jnp
from jax import lax
from jax.experimental import pallas as pl
from jax.experimental.pallas import tpu as pltpu
from jax.experimental.pallas import tpu_sc as plsc

N_POINTS = 16384
N_FEAT = 64
N_CLUSTERS = 128
BLK = 2048
BLK_SHIFT = 11
N_BLK = N_POINTS // BLK

_WORKERS = 8
_ROWS_PER_WORKER = N_CLUSTERS // _WORKERS


def _argmin_body(xt_ref, ct_ref, idx_ref, cand_ref, bv_ref, bi_ref):
    j = pl.program_id(0)
    xbt = xt_ref[0]
    ctb = ct_ref[...]
    dot = lax.dot_general(
        ctb,
        xbt,
        (((0,), (0,)), ((), ())),
        preferred_element_type=jnp.float32,
        precision=lax.Precision.HIGHEST,
    )
    xn = jnp.sum(xbt * xbt, axis=0, keepdims=True)
    s = xn - 2.0 * dot
    bmin = jnp.min(s, axis=1, keepdims=True)
    ii = lax.broadcasted_iota(jnp.int32, s.shape, 1) + j * BLK
    bidx = jnp.min(
        jnp.where(s == bmin, ii, jnp.int32(2**30)), axis=1, keepdims=True
    )
    bidx_t = jnp.transpose(bidx)

    iic = lax.broadcasted_iota(jnp.int32, (BLK, N_CLUSTERS), 0) + j * BLK
    onehot = (iic == bidx_t).astype(jnp.float32)
    sel = lax.dot_general(
        onehot,
        xbt,
        (((0,), (1,)), ((), ())),
        preferred_element_type=jnp.float32,
        precision=lax.Precision.HIGHEST,
    )
    cand_ref[...] = jnp.concatenate([sel, sel], axis=1)

    @pl.when(j == 0)
    def _():
        bv_ref[...] = bmin
        bi_ref[...] = bidx

    @pl.when(j > 0)
    def _():
        better = bmin < bv_ref[...]
        bv_ref[...] = jnp.where(better, bmin, bv_ref[...])
        bi_ref[...] = jnp.where(better, bidx, bi_ref[...])

    @pl.when(j == N_BLK - 1)
    def _():
        idx_ref[...] = jnp.transpose(bi_ref[...])


def _tc_argmin(xt, ct):
    return pl.pallas_call(
        _argmin_body,
        grid=(N_BLK,),
        in_specs=[
            pl.BlockSpec((1, N_FEAT, BLK), lambda j: (0, 0, j)),
            pl.BlockSpec((N_FEAT, N_CLUSTERS), lambda j: (0, 0)),
        ],
        out_specs=[
            pl.BlockSpec((1, N_CLUSTERS), lambda j: (0, 0)),
            pl.BlockSpec((N_CLUSTERS, 2 * N_FEAT), lambda j: (j, 0)),
        ],
        out_shape=[
            jax.ShapeDtypeStruct((1, N_CLUSTERS), jnp.int32),
            jax.ShapeDtypeStruct((N_BLK * N_CLUSTERS, 2 * N_FEAT), jnp.float32),
        ],
        scratch_shapes=[
            pltpu.VMEM((N_CLUSTERS, 1), jnp.float32),
            pltpu.VMEM((N_CLUSTERS, 1), jnp.int32),
        ],
    )(xt, ct)


@functools.lru_cache(maxsize=1)
def _make_sc_gather():
    nc = plsc.get_sparse_core_info().num_cores

    @functools.partial(
        pl.kernel,
        mesh=plsc.VectorSubcoreMesh(core_axis_name="c", subcore_axis_name="s"),
        out_type=jax.ShapeDtypeStruct((1, N_CLUSTERS, N_FEAT), jnp.float32),
        scratch_types=[
            pltpu.VMEM((_ROWS_PER_WORKER,), jnp.int32),
            pltpu.VMEM((_ROWS_PER_WORKER, 2 * N_FEAT), jnp.float32),
            pltpu.VMEM((_ROWS_PER_WORKER, N_FEAT), jnp.float32),
            pltpu.SemaphoreType.DMA,
        ],
    )
    def _sc_gather(cand_hbm, idx_hbm, out_hbm, idx_v, rows_v, out_v, sem):
        wid = lax.axis_index("s") * nc + lax.axis_index("c")

        @pl.when(wid < _WORKERS)
        def _():
            base = wid * _ROWS_PER_WORKER
            pltpu.sync_copy(idx_hbm.at[0, pl.ds(base, _ROWS_PER_WORKER)], idx_v)
            idx16 = idx_v[...]
            lane = lax.broadcasted_iota(jnp.int32, (16,), 0)
            rvec = ((idx16 >> BLK_SHIFT) << 7) + base + lane
            pltpu.async_copy(cand_hbm.at[rvec], rows_v, sem).wait()
            for r in range(_ROWS_PER_WORKER):
                for c4 in range(N_FEAT // 16):
                    out_v[r, pl.ds(16 * c4, 16)] = rows_v[r, pl.ds(16 * c4, 16)]
            pltpu.sync_copy(out_v, out_hbm.at[0, pl.ds(base, _ROWS_PER_WORKER)])

    return _sc_gather


def kernel(x, cluster_centers):
    xt = jnp.swapaxes(x, 1, 2)
    ct = cluster_centers.T
    idx, cands = _tc_argmin(xt, ct)
    return _make_sc_gather()(cands, idx)


# speedup vs baseline: 1.4121x; 1.4121x over previous
import functools

import jax
import jax.numpy as jnp
from jax import lax
from jax.experimental import pallas as pl
from jax.experimental.pallas import tpu as pltpu
from jax.experimental.pallas import tpu_sc as plsc

N_POINTS = 16384
N_FEAT = 64
N_CLUSTERS = 128
BLK = 4096
N_BLK = N_POINTS // BLK

_WORKERS = 8
_ROWS_PER_WORKER = N_CLUSTERS // _WORKERS


def _argmin_body(xt_ref, ct_ref, idx_ref, bv_ref, bi_ref):
    j = pl.program_id(0)
    xbt = xt_ref[0]
    ctb = ct_ref[...]
    dot = lax.dot_general(
        ctb,
        xbt,
        (((0,), (0,)), ((), ())),
        preferred_element_type=jnp.float32,
        precision=lax.Precision.HIGHEST,
    )
    xn = jnp.sum(xbt * xbt, axis=0, keepdims=True)
    s = xn - 2.0 * dot
    bmin = jnp.min(s, axis=1, keepdims=True)
    ii = lax.broadcasted_iota(jnp.int32, s.shape, 1) + j * BLK
    bidx = jnp.min(
        jnp.where(s == bmin, ii, jnp.int32(2**30)), axis=1, keepdims=True
    )

    @pl.when(j == 0)
    def _():
        bv_ref[...] = bmin
        bi_ref[...] = bidx

    @pl.when(j > 0)
    def _():
        better = bmin < bv_ref[...]
        bv_ref[...] = jnp.where(better, bmin, bv_ref[...])
        bi_ref[...] = jnp.where(better, bidx, bi_ref[...])

    @pl.when(j == N_BLK - 1)
    def _():
        idx_ref[...] = jnp.transpose(bi_ref[...])


def _tc_argmin(xt, ct):
    return pl.pallas_call(
        _argmin_body,
        grid=(N_BLK,),
        in_specs=[
            pl.BlockSpec((1, N_FEAT, BLK), lambda j: (0, 0, j)),
            pl.BlockSpec((N_FEAT, N_CLUSTERS), lambda j: (0, 0)),
        ],
        out_specs=pl.BlockSpec((1, N_CLUSTERS), lambda j: (0, 0)),
        out_shape=jax.ShapeDtypeStruct((1, N_CLUSTERS), jnp.int32),
        scratch_shapes=[
            pltpu.VMEM((N_CLUSTERS, 1), jnp.float32),
            pltpu.VMEM((N_CLUSTERS, 1), jnp.int32),
        ],
    )(xt, ct)


@functools.lru_cache(maxsize=1)
def _make_sc_gather():
    nc = plsc.get_sparse_core_info().num_cores

    @functools.partial(
        pl.kernel,
        mesh=plsc.VectorSubcoreMesh(core_axis_name="c", subcore_axis_name="s"),
        out_type=jax.ShapeDtypeStruct((1, N_CLUSTERS, N_FEAT), jnp.float32),
        scratch_types=[
            pltpu.VMEM((_ROWS_PER_WORKER,), jnp.int32),
            pltpu.VMEM((_ROWS_PER_WORKER, N_FEAT), jnp.float32),
            pltpu.SemaphoreType.DMA,
        ],
    )
    def _sc_gather(x_hbm, idx_hbm, out_hbm, idx_v, rows_v, sem):
        wid = lax.axis_index("s") * nc + lax.axis_index("c")

        @pl.when(wid < _WORKERS)
        def _():
            base = wid * _ROWS_PER_WORKER
            pltpu.sync_copy(idx_hbm.at[0, pl.ds(base, _ROWS_PER_WORKER)], idx_v)
            idx16 = idx_v[...]
            copies = []
            for r in range(_ROWS_PER_WORKER):
                i = idx16[r]
                copies.append(
                    pltpu.async_copy(
                        x_hbm.at[0, pl.ds(i, 1), :], rows_v.at[pl.ds(r, 1), :], sem
                    )
                )
            for c in copies:
                c.wait()
            pltpu.sync_copy(rows_v, out_hbm.at[0, pl.ds(base, _ROWS_PER_WORKER)])

    return _sc_gather


def kernel(x, cluster_centers):
    xt = jnp.swapaxes(x, 1, 2)
    ct = cluster_centers.T
    idx = _tc_argmin(xt, ct)
    return _make_sc_gather()(x, idx)


# speedup vs baseline: 1.4301x; 1.0128x over previous
import functools

import jax
import jax.numpy as jnp
from jax import lax
from jax.experimental import pallas as pl
from jax.experimental.pallas import tpu as pltpu
from jax.experimental.pallas import tpu_sc as plsc

N_POINTS = 16384
N_FEAT = 64
N_CLUSTERS = 128
BLK = 8192
N_BLK = N_POINTS // BLK

_WORKERS = 8
_ROWS_PER_WORKER = N_CLUSTERS // _WORKERS


def _argmin_body(xt_ref, ct_ref, idx_ref, bv_ref, bi_ref):
    j = pl.program_id(0)
    xbt = xt_ref[0]
    ctb = ct_ref[...]
    dot = lax.dot_general(
        ctb,
        xbt,
        (((0,), (0,)), ((), ())),
        preferred_element_type=jnp.float32,
        precision=lax.Precision.HIGHEST,
    )
    xn = jnp.sum(xbt * xbt, axis=0, keepdims=True)
    s = xn - 2.0 * dot
    bmin = jnp.min(s, axis=1, keepdims=True)
    ii = lax.broadcasted_iota(jnp.int32, s.shape, 1) + j * BLK
    bidx = jnp.min(
        jnp.where(s == bmin, ii, jnp.int32(2**30)), axis=1, keepdims=True
    )

    @pl.when(j == 0)
    def _():
        bv_ref[...] = bmin
        bi_ref[...] = bidx

    @pl.when(j > 0)
    def _():
        better = bmin < bv_ref[...]
        bv_ref[...] = jnp.where(better, bmin, bv_ref[...])
        bi_ref[...] = jnp.where(better, bidx, bi_ref[...])

    @pl.when(j == N_BLK - 1)
    def _():
        idx_ref[...] = jnp.transpose(bi_ref[...])


def _tc_argmin(xt, ct):
    return pl.pallas_call(
        _argmin_body,
        grid=(N_BLK,),
        in_specs=[
            pl.BlockSpec((1, N_FEAT, BLK), lambda j: (0, 0, j)),
            pl.BlockSpec((N_FEAT, N_CLUSTERS), lambda j: (0, 0)),
        ],
        out_specs=pl.BlockSpec((1, N_CLUSTERS), lambda j: (0, 0)),
        out_shape=jax.ShapeDtypeStruct((1, N_CLUSTERS), jnp.int32),
        scratch_shapes=[
            pltpu.VMEM((N_CLUSTERS, 1), jnp.float32),
            pltpu.VMEM((N_CLUSTERS, 1), jnp.int32),
        ],
    )(xt, ct)


@functools.lru_cache(maxsize=1)
def _make_sc_gather():
    nc = plsc.get_sparse_core_info().num_cores

    @functools.partial(
        pl.kernel,
        mesh=plsc.VectorSubcoreMesh(core_axis_name="c", subcore_axis_name="s"),
        out_type=jax.ShapeDtypeStruct((1, N_CLUSTERS, N_FEAT), jnp.float32),
        scratch_types=[
            pltpu.VMEM((_ROWS_PER_WORKER,), jnp.int32),
            pltpu.VMEM((_ROWS_PER_WORKER, N_FEAT), jnp.float32),
            pltpu.SemaphoreType.DMA,
        ],
    )
    def _sc_gather(x_hbm, idx_hbm, out_hbm, idx_v, rows_v, sem):
        wid = lax.axis_index("s") * nc + lax.axis_index("c")

        @pl.when(wid < _WORKERS)
        def _():
            base = wid * _ROWS_PER_WORKER
            pltpu.sync_copy(idx_hbm.at[0, pl.ds(base, _ROWS_PER_WORKER)], idx_v)
            idx16 = idx_v[...]
            copies = []
            for r in range(_ROWS_PER_WORKER):
                i = idx16[r]
                copies.append(
                    pltpu.async_copy(
                        x_hbm.at[0, pl.ds(i, 1), :], rows_v.at[pl.ds(r, 1), :], sem
                    )
                )
            for c in copies:
                c.wait()
            pltpu.sync_copy(rows_v, out_hbm.at[0, pl.ds(base, _ROWS_PER_WORKER)])

    return _sc_gather


def kernel(x, cluster_centers):
    xt = jnp.swapaxes(x, 1, 2)
    ct = cluster_centers.T
    idx = _tc_argmin(xt, ct)
    return _make_sc_gather()(x, idx)
